# Initial kernel scaffold; baseline (speedup 1.0000x reference)
import jax
import jax.numpy as jnp
from jax.experimental import pallas as pl


def kernel(bond_features, bond_pairs, bond_neighbors, atom_neighbors, xyz, W_ib, b_ib, W_m, b_m, W_hm, b_hm):
    raise NotImplementedError("write your pallas kernel here")


# speedup vs baseline: 19.6425x; 19.6425x over previous
import functools

import jax
import jax.numpy as jnp
from jax import lax
from jax.experimental import pallas as pl
from jax.experimental.pallas import tpu as pltpu
from jax.experimental.pallas import tpu_sc as plsc

E = 160000
H = 64
K = 8
NW = 32
EPW = E // NW
L = 16

EPW_PAD = ((EPW + L - 1) // L) * L
E_PAD = EPW_PAD * NW
_DIST_CHUNKS = EPW_PAD // L

_sc_mesh = plsc.VectorSubcoreMesh(core_axis_name="c", subcore_axis_name="s")
_sc_params = pltpu.CompilerParams(
    needs_layout_passes=False, use_tc_tiling_on_sc=False)


@functools.partial(
    pl.kernel,
    out_type=jax.ShapeDtypeStruct((E_PAD,), jnp.float32),
    mesh=_sc_mesh,
    scratch_types=[
        pltpu.VMEM((10000, 4), jnp.float32),
        pltpu.VMEM((EPW_PAD,), jnp.int32),
        pltpu.VMEM((EPW_PAD,), jnp.int32),
        pltpu.VMEM((EPW_PAD,), jnp.float32),
    ],
    compiler_params=_sc_params,
)
def _dist_call(xyz_hbm, p0_hbm, p1_hbm, d_hbm, xyz_v, p0_v, p1_v, d_v):
    wid = lax.axis_index("s") * 2 + lax.axis_index("c")
    base = wid * EPW_PAD
    pltpu.sync_copy(xyz_hbm, xyz_v)
    pltpu.sync_copy(p0_hbm.at[pl.ds(base, EPW_PAD)], p0_v)
    pltpu.sync_copy(p1_hbm.at[pl.ds(base, EPW_PAD)], p1_v)

    def body(i, carry):
        off = i * L
        a = p0_v[pl.ds(off, L)]
        b = p1_v[pl.ds(off, L)]
        c0 = jnp.zeros((L,), jnp.int32)
        c1 = c0 + 1
        c2 = c0 + 2
        dx = plsc.load_gather(xyz_v, [a, c0]) - plsc.load_gather(xyz_v, [b, c0])
        dy = plsc.load_gather(xyz_v, [a, c1]) - plsc.load_gather(xyz_v, [b, c1])
        dz = plsc.load_gather(xyz_v, [a, c2]) - plsc.load_gather(xyz_v, [b, c2])
        d2 = dx * dx + dy * dy + dz * dz
        r = 1.0 / d2
        r = jnp.where(r == jnp.inf, jnp.zeros_like(r), r)
        d_v[pl.ds(off, L)] = r
        return carry

    lax.fori_loop(0, _DIST_CHUNKS, body, 0)
    pltpu.sync_copy(d_v, d_hbm.at[pl.ds(base, EPW_PAD)])


_GC = 10
_GN = EPW // _GC


@functools.partial(
    pl.kernel,
    out_type=jax.ShapeDtypeStruct((E, H), jnp.float32),
    mesh=_sc_mesh,
    scratch_types=[
        pltpu.VMEM((_GC * K,), jnp.int32),
        pltpu.VMEM((_GC * K, H), jnp.float32),
        pltpu.VMEM((_GC, H), jnp.float32),
        pltpu.SemaphoreType.DMA,
    ],
    compiler_params=_sc_params,
)
def _gather_call(w_hbm, nbr_hbm, msg_hbm, idx_v, rows_v, out_v, sem):
    wid = lax.axis_index("s") * 2 + lax.axis_index("c")
    wbase = wid * EPW

    def body(i, carry):
        base = wbase + i * _GC
        pltpu.sync_copy(nbr_hbm.at[pl.ds(base * K, _GC * K)], idx_v)
        pltpu.async_copy(w_hbm.at[idx_v], rows_v, sem).wait()
        for e in range(_GC):
            for q in range(H // L):
                sl = pl.ds(q * L, L)
                acc = rows_v[e * K, sl]
                for k in range(1, K):
                    acc = acc + rows_v[e * K + k, sl]
                out_v[e, sl] = acc
        pltpu.sync_copy(out_v, msg_hbm.at[pl.ds(base, _GC)])
        return carry

    lax.fori_loop(0, _GN, body, 0)


_TB = 1280
_TGRID = E // _TB

_row_spec = pl.BlockSpec((_TB, H), lambda i: (i, 0))
_d_spec = pl.BlockSpec((_TB, 1), lambda i: (i, 0))
_w_spec = pl.BlockSpec((H, H), lambda i: (0, 0))
_b_spec = pl.BlockSpec((1, H), lambda i: (0, 0))


def _init_body(bf_ref, wib_ref, bib_ref, d_ref, h_ref, w_ref):
    x = jnp.dot(bf_ref[...], wib_ref[...], preferred_element_type=jnp.float32)
    h = jnp.maximum(x + bib_ref[...], 0.0)
    h_ref[...] = h
    w_ref[...] = h * d_ref[...]


_init_call = pl.pallas_call(
    _init_body,
    grid=(_TGRID,),
    in_specs=[_row_spec, _w_spec, _b_spec, _d_spec],
    out_specs=[_row_spec, _row_spec],
    out_shape=[
        jax.ShapeDtypeStruct((E, H), jnp.float32),
        jax.ShapeDtypeStruct((E, H), jnp.float32),
    ],
)


def _upd_core(msg_ref, h_ref, wm_ref, bm_ref, whm_ref, bhm_ref):
    m = jnp.dot(msg_ref[...], wm_ref[...], preferred_element_type=jnp.float32)
    m = jnp.maximum(m + bm_ref[...], 0.0)
    u = h_ref[...] + m
    hn = jnp.dot(u, whm_ref[...], preferred_element_type=jnp.float32)
    return jnp.maximum(hn + bhm_ref[...], 0.0)


def _upd_body(msg_ref, h_ref, d_ref, wm_ref, bm_ref, whm_ref, bhm_ref,
              hn_ref, wn_ref):
    hn = _upd_core(msg_ref, h_ref, wm_ref, bm_ref, whm_ref, bhm_ref)
    hn_ref[...] = hn
    wn_ref[...] = hn * d_ref[...]


_upd_call = pl.pallas_call(
    _upd_body,
    grid=(_TGRID,),
    in_specs=[_row_spec, _row_spec, _d_spec, _w_spec, _b_spec, _w_spec, _b_spec],
    out_specs=[_row_spec, _row_spec],
    out_shape=[
        jax.ShapeDtypeStruct((E, H), jnp.float32),
        jax.ShapeDtypeStruct((E, H), jnp.float32),
    ],
)


def _upd_final_body(msg_ref, h_ref, wm_ref, bm_ref, whm_ref, bhm_ref, hn_ref):
    hn_ref[...] = _upd_core(msg_ref, h_ref, wm_ref, bm_ref, whm_ref, bhm_ref)


_upd_final_call = pl.pallas_call(
    _upd_final_body,
    grid=(_TGRID,),
    in_specs=[_row_spec, _row_spec, _w_spec, _b_spec, _w_spec, _b_spec],
    out_specs=_row_spec,
    out_shape=jax.ShapeDtypeStruct((E, H), jnp.float32),
)


def kernel(bond_features, bond_pairs, bond_neighbors, atom_neighbors, xyz,
           W_ib, b_ib, W_m, b_m, W_hm, b_hm):
    bf = bond_features[0]
    nbr = bond_neighbors[0].reshape(E * K)
    p0 = bond_pairs[0, :, 0]
    p1 = bond_pairs[0, :, 1]
    pad = E_PAD - E
    p0 = jnp.concatenate([p0, jnp.zeros((pad,), jnp.int32)])
    p1 = jnp.concatenate([p1, jnp.zeros((pad,), jnp.int32)])
    xyz_pad = jnp.concatenate(
        [xyz[0], jnp.zeros((10000, 1), jnp.float32)], axis=1)

    d_pad = _dist_call(xyz_pad, p0, p1)
    d = d_pad[:E].reshape(E, 1)

    bm = b_m.reshape(1, H)
    bhm = b_hm.reshape(1, H)
    h, w = _init_call(bf, W_ib, b_ib.reshape(1, H), d)
    for _ in range(3):
        msg = _gather_call(w, nbr)
        h, w = _upd_call(msg, h, d, W_m, bm, W_hm, bhm)
    msg = _gather_call(w, nbr)
    h = _upd_final_call(msg, h, W_m, bm, W_hm, bhm)
    return h.reshape(1, 1, E, H)


# speedup vs baseline: 37.1644x; 1.8920x over previous
import functools

import jax
import jax.numpy as jnp
from jax import lax
from jax.experimental import pallas as pl
from jax.experimental.pallas import tpu as pltpu
from jax.experimental.pallas import tpu_sc as plsc

E = 160000
H = 64
K = 8
NW = 32
EPW = E // NW
L = 16

EPW_PAD = ((EPW + L - 1) // L) * L
E_PAD = EPW_PAD * NW
_DIST_CHUNKS = EPW_PAD // L

_sc_mesh = plsc.VectorSubcoreMesh(core_axis_name="c", subcore_axis_name="s")
_sc_params = pltpu.CompilerParams(
    needs_layout_passes=False, use_tc_tiling_on_sc=False)


@functools.partial(
    pl.kernel,
    out_type=jax.ShapeDtypeStruct((E_PAD,), jnp.float32),
    mesh=_sc_mesh,
    scratch_types=[
        pltpu.VMEM((10000, 4), jnp.float32),
        pltpu.VMEM((EPW_PAD,), jnp.int32),
        pltpu.VMEM((EPW_PAD,), jnp.int32),
        pltpu.VMEM((EPW_PAD,), jnp.float32),
    ],
    compiler_params=_sc_params,
)
def _dist_call(xyz_hbm, p0_hbm, p1_hbm, d_hbm, xyz_v, p0_v, p1_v, d_v):
    wid = lax.axis_index("s") * 2 + lax.axis_index("c")
    base = wid * EPW_PAD
    pltpu.sync_copy(xyz_hbm, xyz_v)
    pltpu.sync_copy(p0_hbm.at[pl.ds(base, EPW_PAD)], p0_v)
    pltpu.sync_copy(p1_hbm.at[pl.ds(base, EPW_PAD)], p1_v)

    def body(i, carry):
        off = i * L
        a = p0_v[pl.ds(off, L)]
        b = p1_v[pl.ds(off, L)]
        c0 = jnp.zeros((L,), jnp.int32)
        c1 = c0 + 1
        c2 = c0 + 2
        dx = plsc.load_gather(xyz_v, [a, c0]) - plsc.load_gather(xyz_v, [b, c0])
        dy = plsc.load_gather(xyz_v, [a, c1]) - plsc.load_gather(xyz_v, [b, c1])
        dz = plsc.load_gather(xyz_v, [a, c2]) - plsc.load_gather(xyz_v, [b, c2])
        d2 = dx * dx + dy * dy + dz * dz
        r = 1.0 / d2
        r = jnp.where(r == jnp.inf, jnp.zeros_like(r), r)
        d_v[pl.ds(off, L)] = r
        return carry

    lax.fori_loop(0, _DIST_CHUNKS, body, 0)
    pltpu.sync_copy(d_v, d_hbm.at[pl.ds(base, EPW_PAD)])


_IR = 80
_NR = EPW * K // _IR
_GG = 5
_CE = _GG * _IR // K
_NG = _NR // _GG


@functools.partial(
    pl.kernel,
    out_type=jax.ShapeDtypeStruct((E, H), jnp.float32),
    mesh=_sc_mesh,
    scratch_types=[
        pltpu.VMEM((_NR, _IR), jnp.int32),
        pltpu.VMEM((_GG * _IR, H), jnp.float32),
        pltpu.VMEM((_GG * _IR, H), jnp.float32),
        pltpu.VMEM((_CE, H), jnp.float32),
        pltpu.VMEM((_CE, H), jnp.float32),
        pltpu.SemaphoreType.DMA,
        pltpu.SemaphoreType.DMA,
        pltpu.SemaphoreType.DMA,
        pltpu.SemaphoreType.DMA,
    ],
    compiler_params=_sc_params,
)
def _gather_call(w_hbm, nbr_hbm, msg_hbm, idx_v, rows_a, rows_b, out_a, out_b,
                 gsem_a, gsem_b, osem_a, osem_b):
    wid = lax.axis_index("s") * 2 + lax.axis_index("c")
    wbase = wid * EPW

    def fire_group(g, rows_v, gsem):
        for j in range(_GG):
            pltpu.async_copy(w_hbm.at[idx_v.at[g * _GG + j]],
                             rows_v.at[pl.ds(j * _IR, _IR)], gsem)

    def wait_group(rows_v, gsem):
        for j in range(_GG):
            pltpu.make_async_copy(w_hbm.at[idx_v.at[0]],
                                  rows_v.at[pl.ds(j * _IR, _IR)], gsem).wait()

    def accumulate(rows_v, out_v):
        def ebody(e, c):
            for q in range(H // L):
                sl = pl.ds(q * L, L)
                acc = rows_v[e * K, sl]
                for k in range(1, K):
                    acc = acc + rows_v[e * K + k, sl]
                out_v[e, sl] = acc
            return c
        lax.fori_loop(0, _CE, ebody, 0)

    pltpu.sync_copy(nbr_hbm.at[pl.ds(wid * _NR, _NR)], idx_v)
    fire_group(0, rows_a, gsem_a)
    fire_group(1, rows_b, gsem_b)

    bufs = ((rows_a, out_a, gsem_a, osem_a), (rows_b, out_b, gsem_b, osem_b))

    def tbody(t, carry):
        for b, (rows_v, out_v, gsem, osem) in enumerate(bufs):
            g = t * 2 + b
            wait_group(rows_v, gsem)

            @pl.when(t > 0)
            def _wait_out():
                pltpu.make_async_copy(
                    out_v, msg_hbm.at[pl.ds(wbase, _CE)], osem).wait()

            accumulate(rows_v, out_v)

            @pl.when(t < _NG // 2 - 1)
            def _fire_next():
                fire_group(g + 2, rows_v, gsem)

            pltpu.async_copy(out_v, msg_hbm.at[pl.ds(wbase + g * _CE, _CE)],
                             osem)
        return carry

    lax.fori_loop(0, _NG // 2, tbody, 0)
    for _, out_v, _, osem in bufs:
        pltpu.make_async_copy(out_v, msg_hbm.at[pl.ds(wbase, _CE)], osem).wait()


_TB = 1280
_TGRID = E // _TB

_row_spec = pl.BlockSpec((_TB, H), lambda i: (i, 0))
_d_spec = pl.BlockSpec((_TB, 1), lambda i: (i, 0))
_w_spec = pl.BlockSpec((H, H), lambda i: (0, 0))
_b_spec = pl.BlockSpec((1, H), lambda i: (0, 0))


def _init_body(bf_ref, wib_ref, bib_ref, d_ref, h_ref, w_ref):
    x = jnp.dot(bf_ref[...], wib_ref[...], preferred_element_type=jnp.float32)
    h = jnp.maximum(x + bib_ref[...], 0.0)
    h_ref[...] = h
    w_ref[...] = h * d_ref[...]


_init_call = pl.pallas_call(
    _init_body,
    grid=(_TGRID,),
    in_specs=[_row_spec, _w_spec, _b_spec, _d_spec],
    out_specs=[_row_spec, _row_spec],
    out_shape=[
        jax.ShapeDtypeStruct((E, H), jnp.float32),
        jax.ShapeDtypeStruct((E, H), jnp.float32),
    ],
)


def _upd_core(msg_ref, h_ref, wm_ref, bm_ref, whm_ref, bhm_ref):
    m = jnp.dot(msg_ref[...], wm_ref[...], preferred_element_type=jnp.float32)
    m = jnp.maximum(m + bm_ref[...], 0.0)
    u = h_ref[...] + m
    hn = jnp.dot(u, whm_ref[...], preferred_element_type=jnp.float32)
    return jnp.maximum(hn + bhm_ref[...], 0.0)


def _upd_body(msg_ref, h_ref, d_ref, wm_ref, bm_ref, whm_ref, bhm_ref,
              hn_ref, wn_ref):
    hn = _upd_core(msg_ref, h_ref, wm_ref, bm_ref, whm_ref, bhm_ref)
    hn_ref[...] = hn
    wn_ref[...] = hn * d_ref[...]


_upd_call = pl.pallas_call(
    _upd_body,
    grid=(_TGRID,),
    in_specs=[_row_spec, _row_spec, _d_spec, _w_spec, _b_spec, _w_spec, _b_spec],
    out_specs=[_row_spec, _row_spec],
    out_shape=[
        jax.ShapeDtypeStruct((E, H), jnp.float32),
        jax.ShapeDtypeStruct((E, H), jnp.float32),
    ],
)


def _upd_final_body(msg_ref, h_ref, wm_ref, bm_ref, whm_ref, bhm_ref, hn_ref):
    hn_ref[...] = _upd_core(msg_ref, h_ref, wm_ref, bm_ref, whm_ref, bhm_ref)


_upd_final_call = pl.pallas_call(
    _upd_final_body,
    grid=(_TGRID,),
    in_specs=[_row_spec, _row_spec, _w_spec, _b_spec, _w_spec, _b_spec],
    out_specs=_row_spec,
    out_shape=jax.ShapeDtypeStruct((E, H), jnp.float32),
)


def kernel(bond_features, bond_pairs, bond_neighbors, atom_neighbors, xyz,
           W_ib, b_ib, W_m, b_m, W_hm, b_hm):
    bf = bond_features[0]
    nbr = bond_neighbors[0].reshape(E * K // _IR, _IR)
    p0 = bond_pairs[0, :, 0]
    p1 = bond_pairs[0, :, 1]
    pad = E_PAD - E
    p0 = jnp.concatenate([p0, jnp.zeros((pad,), jnp.int32)])
    p1 = jnp.concatenate([p1, jnp.zeros((pad,), jnp.int32)])
    xyz_pad = jnp.concatenate(
        [xyz[0], jnp.zeros((10000, 1), jnp.float32)], axis=1)

    d_pad = _dist_call(xyz_pad, p0, p1)
    d = d_pad[:E].reshape(E, 1)

    bm = b_m.reshape(1, H)
    bhm = b_hm.reshape(1, H)
    h, w = _init_call(bf, W_ib, b_ib.reshape(1, H), d)
    for _ in range(3):
        msg = _gather_call(w, nbr)
        h, w = _upd_call(msg, h, d, W_m, bm, W_hm, bhm)
    msg = _gather_call(w, nbr)
    h = _upd_final_call(msg, h, W_m, bm, W_hm, bhm)
    return h.reshape(1, 1, E, H)


# speedup vs baseline: 37.2197x; 1.0015x over previous
import functools

import jax
import jax.numpy as jnp
from jax import lax
from jax.experimental import pallas as pl
from jax.experimental.pallas import tpu as pltpu
from jax.experimental.pallas import tpu_sc as plsc

E = 160000
H = 64
K = 8
NW = 32
EPW = E // NW
L = 16

EPW_PAD = ((EPW + L - 1) // L) * L
E_PAD = EPW_PAD * NW
_DIST_CHUNKS = EPW_PAD // L

_sc_mesh = plsc.VectorSubcoreMesh(core_axis_name="c", subcore_axis_name="s")
_sc_params = pltpu.CompilerParams(
    needs_layout_passes=False, use_tc_tiling_on_sc=False)


@functools.partial(
    pl.kernel,
    out_type=jax.ShapeDtypeStruct((E_PAD,), jnp.float32),
    mesh=_sc_mesh,
    scratch_types=[
        pltpu.VMEM((10000, 4), jnp.float32),
        pltpu.VMEM((EPW_PAD,), jnp.int32),
        pltpu.VMEM((EPW_PAD,), jnp.int32),
        pltpu.VMEM((EPW_PAD,), jnp.float32),
    ],
    compiler_params=_sc_params,
)
def _dist_call(xyz_hbm, p0_hbm, p1_hbm, d_hbm, xyz_v, p0_v, p1_v, d_v):
    wid = lax.axis_index("s") * 2 + lax.axis_index("c")
    base = wid * EPW_PAD
    pltpu.sync_copy(xyz_hbm, xyz_v)
    pltpu.sync_copy(p0_hbm.at[pl.ds(base, EPW_PAD)], p0_v)
    pltpu.sync_copy(p1_hbm.at[pl.ds(base, EPW_PAD)], p1_v)

    def body(i, carry):
        off = i * L
        a = p0_v[pl.ds(off, L)]
        b = p1_v[pl.ds(off, L)]
        c0 = jnp.zeros((L,), jnp.int32)
        c1 = c0 + 1
        c2 = c0 + 2
        dx = plsc.load_gather(xyz_v, [a, c0]) - plsc.load_gather(xyz_v, [b, c0])
        dy = plsc.load_gather(xyz_v, [a, c1]) - plsc.load_gather(xyz_v, [b, c1])
        dz = plsc.load_gather(xyz_v, [a, c2]) - plsc.load_gather(xyz_v, [b, c2])
        d2 = dx * dx + dy * dy + dz * dz
        r = 1.0 / d2
        r = jnp.where(r == jnp.inf, jnp.zeros_like(r), r)
        d_v[pl.ds(off, L)] = r
        return carry

    lax.fori_loop(0, _DIST_CHUNKS, body, 0)
    pltpu.sync_copy(d_v, d_hbm.at[pl.ds(base, EPW_PAD)])


_IR = 80
_NR = EPW * K // _IR
_GG = 5
_CE = _GG * _IR // K
_NG = _NR // _GG


@functools.partial(
    pl.kernel,
    out_type=jax.ShapeDtypeStruct((E // 2, 2 * H), jnp.float32),
    mesh=_sc_mesh,
    scratch_types=[
        pltpu.VMEM((_NR, _IR), jnp.int32),
        pltpu.VMEM((_GG * _IR, H), jnp.float32),
        pltpu.VMEM((_GG * _IR, H), jnp.float32),
        pltpu.VMEM((_CE // 2, 2 * H), jnp.float32),
        pltpu.VMEM((_CE // 2, 2 * H), jnp.float32),
        pltpu.SemaphoreType.DMA,
        pltpu.SemaphoreType.DMA,
        pltpu.SemaphoreType.DMA,
        pltpu.SemaphoreType.DMA,
    ],
    compiler_params=_sc_params,
)
def _gather_call(w_hbm, nbr_hbm, msg_hbm, idx_v, rows_a, rows_b, out_a, out_b,
                 gsem_a, gsem_b, osem_a, osem_b):
    wid = lax.axis_index("s") * 2 + lax.axis_index("c")
    wrow = wid * (EPW // 2)
    _CR = _CE // 2

    def fire_group(g, rows_v, gsem):
        for j in range(_GG):
            pltpu.async_copy(w_hbm.at[idx_v.at[g * _GG + j]],
                             rows_v.at[pl.ds(j * _IR, _IR)], gsem)

    def wait_group(rows_v, gsem):
        for j in range(_GG):
            pltpu.make_async_copy(w_hbm.at[idx_v.at[0]],
                                  rows_v.at[pl.ds(j * _IR, _IR)], gsem).wait()

    def accumulate(rows_v, out_v):
        def ebody(e2, c):
            for p in range(2):
                for q in range(H // L):
                    sl = pl.ds(p * H + q * L, L)
                    acc = rows_v[(e2 * 2 + p) * K, pl.ds(q * L, L)]
                    for k in range(1, K):
                        acc = acc + rows_v[(e2 * 2 + p) * K + k, pl.ds(q * L, L)]
                    out_v[e2, sl] = acc
            return c
        lax.fori_loop(0, _CE // 2, ebody, 0)

    pltpu.sync_copy(nbr_hbm.at[pl.ds(wid * _NR, _NR)], idx_v)
    fire_group(0, rows_a, gsem_a)
    fire_group(1, rows_b, gsem_b)

    bufs = ((rows_a, out_a, gsem_a, osem_a), (rows_b, out_b, gsem_b, osem_b))

    def tbody(t, carry):
        for b, (rows_v, out_v, gsem, osem) in enumerate(bufs):
            g = t * 2 + b
            wait_group(rows_v, gsem)

            @pl.when(t > 0)
            def _wait_out():
                pltpu.make_async_copy(
                    out_v, msg_hbm.at[pl.ds(wrow, _CR)], osem).wait()

            accumulate(rows_v, out_v)

            @pl.when(t < _NG // 2 - 1)
            def _fire_next():
                fire_group(g + 2, rows_v, gsem)

            pltpu.async_copy(out_v, msg_hbm.at[pl.ds(wrow + g * _CR, _CR)],
                             osem)
        return carry

    lax.fori_loop(0, _NG // 2, tbody, 0)
    for _, out_v, _, osem in bufs:
        pltpu.make_async_copy(out_v, msg_hbm.at[pl.ds(wrow, _CR)], osem).wait()


_TB = 1280
_TGRID = E // _TB

_row_spec = pl.BlockSpec((_TB, H), lambda i: (i, 0))
_d_spec = pl.BlockSpec((_TB, 1), lambda i: (i, 0))
_w_spec = pl.BlockSpec((H, H), lambda i: (0, 0))
_b_spec = pl.BlockSpec((1, H), lambda i: (0, 0))


def _init_body(bf_ref, wib_ref, bib_ref, d_ref, h_ref, w_ref):
    x = jnp.dot(bf_ref[...], wib_ref[...], preferred_element_type=jnp.float32)
    h = jnp.maximum(x + bib_ref[...], 0.0)
    h_ref[...] = h
    w_ref[...] = h * d_ref[...]


_init_call = pl.pallas_call(
    _init_body,
    grid=(_TGRID,),
    in_specs=[_row_spec, _w_spec, _b_spec, _d_spec],
    out_specs=[_row_spec, _row_spec],
    out_shape=[
        jax.ShapeDtypeStruct((E, H), jnp.float32),
        jax.ShapeDtypeStruct((E, H), jnp.float32),
    ],
)


def _upd_core(msg_ref, h_ref, wm_ref, bm_ref, whm_ref, bhm_ref):
    m = jnp.dot(msg_ref[...], wm_ref[...], preferred_element_type=jnp.float32)
    m = jnp.maximum(m + bm_ref[...], 0.0)
    u = h_ref[...] + m
    hn = jnp.dot(u, whm_ref[...], preferred_element_type=jnp.float32)
    return jnp.maximum(hn + bhm_ref[...], 0.0)


def _upd_body(msg_ref, h_ref, d_ref, wm_ref, bm_ref, whm_ref, bhm_ref,
              hn_ref, wn_ref):
    hn = _upd_core(msg_ref, h_ref, wm_ref, bm_ref, whm_ref, bhm_ref)
    hn_ref[...] = hn
    wn_ref[...] = hn * d_ref[...]


_upd_call = pl.pallas_call(
    _upd_body,
    grid=(_TGRID,),
    in_specs=[_row_spec, _row_spec, _d_spec, _w_spec, _b_spec, _w_spec, _b_spec],
    out_specs=[_row_spec, _row_spec],
    out_shape=[
        jax.ShapeDtypeStruct((E, H), jnp.float32),
        jax.ShapeDtypeStruct((E, H), jnp.float32),
    ],
)


def _upd_final_body(msg_ref, h_ref, wm_ref, bm_ref, whm_ref, bhm_ref, hn_ref):
    hn_ref[...] = _upd_core(msg_ref, h_ref, wm_ref, bm_ref, whm_ref, bhm_ref)


_upd_final_call = pl.pallas_call(
    _upd_final_body,
    grid=(_TGRID,),
    in_specs=[_row_spec, _row_spec, _w_spec, _b_spec, _w_spec, _b_spec],
    out_specs=_row_spec,
    out_shape=jax.ShapeDtypeStruct((E, H), jnp.float32),
)


def kernel(bond_features, bond_pairs, bond_neighbors, atom_neighbors, xyz,
           W_ib, b_ib, W_m, b_m, W_hm, b_hm):
    bf = bond_features[0]
    nbr = bond_neighbors[0].reshape(E * K // _IR, _IR)
    p0 = bond_pairs[0, :, 0]
    p1 = bond_pairs[0, :, 1]
    pad = E_PAD - E
    p0 = jnp.concatenate([p0, jnp.zeros((pad,), jnp.int32)])
    p1 = jnp.concatenate([p1, jnp.zeros((pad,), jnp.int32)])
    xyz_pad = jnp.concatenate(
        [xyz[0], jnp.zeros((10000, 1), jnp.float32)], axis=1)

    d_pad = _dist_call(xyz_pad, p0, p1)
    d = d_pad[:E].reshape(E, 1)

    bm = b_m.reshape(1, H)
    bhm = b_hm.reshape(1, H)
    h, w = _init_call(bf, W_ib, b_ib.reshape(1, H), d)
    for _ in range(3):
        msg = _gather_call(w, nbr).reshape(E, H)
        h, w = _upd_call(msg, h, d, W_m, bm, W_hm, bhm)
    msg = _gather_call(w, nbr).reshape(E, H)
    h = _upd_final_call(msg, h, W_m, bm, W_hm, bhm)
    return h.reshape(1, 1, E, H)
